# Initial kernel scaffold; baseline (speedup 1.0000x reference)
import jax
import jax.numpy as jnp
from jax.experimental import pallas as pl


def kernel(x, mod_sig, feedback, width, depth, mix):
    raise NotImplementedError("write your pallas kernel here")


# speedup vs baseline: 1001.5580x; 1001.5580x over previous
import functools

import jax
import jax.numpy as jnp
from jax import lax
from jax.experimental import pallas as pl
from jax.experimental.pallas import tpu as pltpu
from jax.experimental.pallas import tpu_sc as plsc

B = 32
C = 2
N = 8192
D = 485
MIN_D = 44
MAX_LFO = 441
PAD = 496
K = 32
L = 16


def _flanger_body(x_hbm, mod_hbm, par_hbm, out_hbm,
                  modv, x0, x1, h0, h1, o0, o1, pvec):
    b = lax.axis_index("s") * 2 + lax.axis_index("c")

    pltpu.sync_copy(mod_hbm.at[b], modv)
    pltpu.sync_copy(x_hbm.at[2 * b], x0)
    pltpu.sync_copy(x_hbm.at[2 * b + 1], x1)
    pltpu.sync_copy(par_hbm.at[b], pvec)

    cw = pvec[pl.ds(0, L)]
    fb = pvec[pl.ds(L, L)]
    dp = pvec[pl.ds(2 * L, L)]
    mx = pvec[pl.ds(3 * L, L)]

    zero = jnp.zeros((L,), jnp.float32)
    for j in range(PAD // L):
        h0[pl.ds(j * L, L)] = zero
        h1[pl.ds(j * L, L)] = zero

    lanes = lax.iota(jnp.int32, L)
    fD = jnp.float32(D)

    def chunk(i, carry):
        t0 = i * K
        for g in range(K // L):
            off = pl.multiple_of(t0 + g * L, L)
            tvec = off + lanes
            wq = tvec % D
            wf = wq.astype(jnp.float32)
            mv = modv[pl.ds(off, L)]
            delay = cw * mv + jnp.float32(MIN_D)
            a = wf - delay + fD
            rid = jnp.where(a >= fD, a - fD, a)
            piv = rid.astype(jnp.int32)
            fr = rid - piv.astype(jnp.float32)
            dist = wq - piv
            dist = jnp.where(dist <= 0, dist + D, dist)
            gidx = tvec - dist + PAD
            for (xr, hr, orr) in ((x0, h0, o0), (x1, h1, o1)):
                pv = plsc.load_gather(hr, [gidx])
                nv = plsc.load_gather(hr, [gidx + 1])
                iv = fr * nv + (jnp.float32(1.0) - fr) * pv
                xv = xr[pl.ds(off, L)]
                hr[pl.ds(off + PAD, L)] = xv + fb * iv
                ov = xv + dp * iv
                orr[pl.ds(off, L)] = (jnp.float32(1.0) - mx) * xv + mx * ov
        return carry

    lax.fori_loop(0, N // K, chunk, 0)

    pltpu.sync_copy(o0, out_hbm.at[2 * b])
    pltpu.sync_copy(o1, out_hbm.at[2 * b + 1])


@jax.jit
def _flanger(x2, mod_sig, params):
    mesh = plsc.VectorSubcoreMesh(core_axis_name="c", subcore_axis_name="s")
    f = functools.partial(
        pl.kernel,
        mesh=mesh,
        compiler_params=pltpu.CompilerParams(needs_layout_passes=False),
        out_type=jax.ShapeDtypeStruct((B * C, N), jnp.float32),
        scratch_types=[
            pltpu.VMEM((N,), jnp.float32),
            pltpu.VMEM((N,), jnp.float32),
            pltpu.VMEM((N,), jnp.float32),
            pltpu.VMEM((PAD + N,), jnp.float32),
            pltpu.VMEM((PAD + N,), jnp.float32),
            pltpu.VMEM((N,), jnp.float32),
            pltpu.VMEM((N,), jnp.float32),
            pltpu.VMEM((4 * L,), jnp.float32),
        ],
    )(_flanger_body)
    return f(x2, mod_sig, params)


def kernel(x, mod_sig, feedback, width, depth, mix):
    x2 = x.reshape(B * C, N)
    cw = (MAX_LFO * width).astype(jnp.float32)
    params = jnp.stack([cw, feedback, depth, mix], axis=1)
    params = jnp.broadcast_to(params[:, :, None], (B, 4, L)).reshape(B, 4 * L)
    out2 = _flanger(x2, mod_sig, params)
    return out2.reshape(B, C, N)


# speedup vs baseline: 1011.0423x; 1.0095x over previous
import functools

import jax
import jax.numpy as jnp
from jax import lax
from jax.experimental import pallas as pl
from jax.experimental.pallas import tpu as pltpu
from jax.experimental.pallas import tpu_sc as plsc

B = 32
C = 2
N = 8192
D = 485
MIN_D = 44
MAX_LFO = 441
PAD = 496
K = 32
L = 16


def _flanger_body(x_hbm, mod_hbm, par_hbm, out_hbm,
                  modv, x0, x1, h0, h1, o0, o1, pvec):
    b = lax.axis_index("s") * 2 + lax.axis_index("c")

    pltpu.sync_copy(mod_hbm.at[b], modv.at[pl.ds(0, N)])
    pltpu.sync_copy(x_hbm.at[2 * b], x0)
    pltpu.sync_copy(x_hbm.at[2 * b + 1], x1)
    pltpu.sync_copy(par_hbm.at[b], pvec)

    cw = pvec[pl.ds(0, L)]
    fb = pvec[pl.ds(L, L)]
    dp = pvec[pl.ds(2 * L, L)]
    mx = pvec[pl.ds(3 * L, L)]
    omx = jnp.float32(1.0) - mx

    zero = jnp.zeros((L,), jnp.float32)
    for j in range(PAD // L):
        h0[pl.ds(j * L, L)] = zero
        h1[pl.ds(j * L, L)] = zero

    lanes = lax.iota(jnp.int32, L)
    fD = jnp.float32(D)
    fMIN = jnp.float32(MIN_D)

    def idxmath(t0, wq0):
        res = []
        for g in range(K // L):
            wq0g = wq0 + g * L
            wq0g = jnp.where(wq0g >= D, wq0g - D, wq0g)
            tp = (t0 + g * L + PAD) + lanes
            wqv = wq0g + lanes
            wqv = jnp.where(wqv >= D, wqv - D, wqv)
            wf = wqv.astype(jnp.float32)
            mv = modv[pl.ds(pl.multiple_of(t0 + g * L, L), L)]
            delay = cw * mv + fMIN
            a = wf - delay + fD
            rid = jnp.where(a >= fD, a - fD, a)
            piv = rid.astype(jnp.int32)
            fr = rid - piv.astype(jnp.float32)
            dist = wqv - piv
            dist = jnp.where(dist <= 0, dist + D, dist)
            res += [tp - dist, fr]
        return res

    def chunk(i, carry):
        wq0n, g0, f0, g1, f1 = carry
        t0 = i * K
        for g, (gi, fr) in enumerate(((g0, f0), (g1, f1))):
            off = pl.multiple_of(t0 + g * L, L)
            gn = gi + 1
            om = jnp.float32(1.0) - fr
            for (xr, hr, orr) in ((x0, h0, o0), (x1, h1, o1)):
                pv = plsc.load_gather(hr, [gi])
                nv = plsc.load_gather(hr, [gn])
                iv = fr * nv + om * pv
                xv = xr[pl.ds(off, L)]
                hr[pl.ds(off + PAD, L)] = xv + fb * iv
                ov = xv + dp * iv
                orr[pl.ds(off, L)] = omx * xv + mx * ov
        nxt = idxmath(t0 + K, wq0n)
        wq0nn = wq0n + K
        wq0nn = jnp.where(wq0nn >= D, wq0nn - D, wq0nn)
        return (wq0nn, *nxt)

    first = idxmath(jnp.int32(0), jnp.int32(0))
    lax.fori_loop(0, N // K, chunk, (jnp.int32(K), *first))

    pltpu.sync_copy(o0, out_hbm.at[2 * b])
    pltpu.sync_copy(o1, out_hbm.at[2 * b + 1])


@jax.jit
def _flanger(x2, mod_sig, params):
    mesh = plsc.VectorSubcoreMesh(core_axis_name="c", subcore_axis_name="s")
    f = functools.partial(
        pl.kernel,
        mesh=mesh,
        compiler_params=pltpu.CompilerParams(needs_layout_passes=False),
        out_type=jax.ShapeDtypeStruct((B * C, N), jnp.float32),
        scratch_types=[
            pltpu.VMEM((N + K,), jnp.float32),
            pltpu.VMEM((N,), jnp.float32),
            pltpu.VMEM((N,), jnp.float32),
            pltpu.VMEM((PAD + N,), jnp.float32),
            pltpu.VMEM((PAD + N,), jnp.float32),
            pltpu.VMEM((N,), jnp.float32),
            pltpu.VMEM((N,), jnp.float32),
            pltpu.VMEM((4 * L,), jnp.float32),
        ],
    )(_flanger_body)
    return f(x2, mod_sig, params)


def kernel(x, mod_sig, feedback, width, depth, mix):
    x2 = x.reshape(B * C, N)
    cw = (MAX_LFO * width).astype(jnp.float32)
    params = jnp.stack([cw, feedback, depth, mix], axis=1)
    params = jnp.broadcast_to(params[:, :, None], (B, 4, L)).reshape(B, 4 * L)
    out2 = _flanger(x2, mod_sig, params)
    return out2.reshape(B, C, N)


# speedup vs baseline: 1137.1913x; 1.1248x over previous
import functools

import jax
import jax.numpy as jnp
from jax import lax
from jax.experimental import pallas as pl
from jax.experimental.pallas import tpu as pltpu
from jax.experimental.pallas import tpu_sc as plsc

B = 32
C = 2
N = 8192
D = 485
MIN_D = 44
MAX_LFO = 441
PAD = 496
K = 32
L = 16


def _flanger_body(x_hbm, mod_hbm, fb_hbm, w_hbm, dp_hbm, mx_hbm, out_hbm,
                  modv, x0, x1, h0, h1, o0, o1, pvec):
    b = lax.axis_index("s") * 2 + lax.axis_index("c")

    pltpu.sync_copy(mod_hbm.at[b], modv.at[pl.ds(0, N)])
    pltpu.sync_copy(x_hbm.at[b, 0], x0)
    pltpu.sync_copy(x_hbm.at[b, 1], x1)
    pltpu.sync_copy(fb_hbm, pvec.at[pl.ds(0, B)])
    pltpu.sync_copy(w_hbm, pvec.at[pl.ds(B, B)])
    pltpu.sync_copy(dp_hbm, pvec.at[pl.ds(2 * B, B)])
    pltpu.sync_copy(mx_hbm, pvec.at[pl.ds(3 * B, B)])

    bvec = jnp.full((L,), b, jnp.int32)
    fb = plsc.load_gather(pvec, [bvec])
    wd = plsc.load_gather(pvec, [bvec + B])
    dp = plsc.load_gather(pvec, [bvec + 2 * B])
    mx = plsc.load_gather(pvec, [bvec + 3 * B])
    cw = jnp.float32(MAX_LFO) * wd
    omx = jnp.float32(1.0) - mx

    zero = jnp.zeros((L,), jnp.float32)
    for j in range(PAD // L):
        h0[pl.ds(j * L, L)] = zero
        h1[pl.ds(j * L, L)] = zero

    lanes = lax.iota(jnp.int32, L)
    fD = jnp.float32(D)
    fMIN = jnp.float32(MIN_D)

    def idxmath(t0, wq0):
        res = []
        for g in range(K // L):
            wq0g = wq0 + g * L
            wq0g = jnp.where(wq0g >= D, wq0g - D, wq0g)
            tp = (t0 + g * L + PAD) + lanes
            wqv = wq0g + lanes
            wqv = jnp.where(wqv >= D, wqv - D, wqv)
            wf = wqv.astype(jnp.float32)
            mv = modv[pl.ds(pl.multiple_of(t0 + g * L, L), L)]
            delay = cw * mv + fMIN
            a = wf - delay + fD
            rid = jnp.where(a >= fD, a - fD, a)
            piv = rid.astype(jnp.int32)
            fr = rid - piv.astype(jnp.float32)
            dist = wqv - piv
            dist = jnp.where(dist <= 0, dist + D, dist)
            res += [tp - dist, fr]
        return res

    def chunk(i, carry):
        wq0n, g0, f0, g1, f1 = carry
        t0 = i * K
        for g, (gi, fr) in enumerate(((g0, f0), (g1, f1))):
            off = pl.multiple_of(t0 + g * L, L)
            gn = gi + 1
            om = jnp.float32(1.0) - fr
            for (xr, hr, orr) in ((x0, h0, o0), (x1, h1, o1)):
                pv = plsc.load_gather(hr, [gi])
                nv = plsc.load_gather(hr, [gn])
                iv = fr * nv + om * pv
                xv = xr[pl.ds(off, L)]
                hr[pl.ds(off + PAD, L)] = xv + fb * iv
                ov = xv + dp * iv
                orr[pl.ds(off, L)] = omx * xv + mx * ov
        nxt = idxmath(t0 + K, wq0n)
        wq0nn = wq0n + K
        wq0nn = jnp.where(wq0nn >= D, wq0nn - D, wq0nn)
        return (wq0nn, *nxt)

    first = idxmath(jnp.int32(0), jnp.int32(0))
    lax.fori_loop(0, N // K, chunk, (jnp.int32(K), *first))

    pltpu.sync_copy(o0, out_hbm.at[b, 0])
    pltpu.sync_copy(o1, out_hbm.at[b, 1])


@jax.jit
def kernel(x, mod_sig, feedback, width, depth, mix):
    mesh = plsc.VectorSubcoreMesh(core_axis_name="c", subcore_axis_name="s")
    f = functools.partial(
        pl.kernel,
        mesh=mesh,
        compiler_params=pltpu.CompilerParams(needs_layout_passes=False),
        out_type=jax.ShapeDtypeStruct((B, C, N), jnp.float32),
        scratch_types=[
            pltpu.VMEM((N + K,), jnp.float32),
            pltpu.VMEM((N,), jnp.float32),
            pltpu.VMEM((N,), jnp.float32),
            pltpu.VMEM((PAD + N,), jnp.float32),
            pltpu.VMEM((PAD + N,), jnp.float32),
            pltpu.VMEM((N,), jnp.float32),
            pltpu.VMEM((N,), jnp.float32),
            pltpu.VMEM((4 * B,), jnp.float32),
        ],
    )(_flanger_body)
    return f(x, mod_sig, feedback, width, depth, mix)


# speedup vs baseline: 1540.5284x; 1.3547x over previous
import functools

import jax
import jax.numpy as jnp
from jax import lax
from jax.experimental import pallas as pl
from jax.experimental.pallas import tpu as pltpu
from jax.experimental.pallas import tpu_sc as plsc

B = 32
C = 2
N = 8192
D = 485
MIN_D = 44
MAX_LFO = 441
PAD = 496
L = 16
NG = N // L


def _flanger_body(x_hbm, mod_hbm, fb_hbm, w_hbm, dp_hbm, mx_hbm, out_hbm,
                  modv, x0, x1, h0, h1, o0, o1, pvec, sem):
    b = lax.axis_index("s") * 2 + lax.axis_index("c")

    cps = [
        pltpu.async_copy(mod_hbm.at[b], modv.at[pl.ds(0, N)], sem),
        pltpu.async_copy(x_hbm.at[b, 0], x0, sem),
        pltpu.async_copy(x_hbm.at[b, 1], x1, sem),
        pltpu.async_copy(fb_hbm, pvec.at[pl.ds(0, B)], sem),
        pltpu.async_copy(w_hbm, pvec.at[pl.ds(B, B)], sem),
        pltpu.async_copy(dp_hbm, pvec.at[pl.ds(2 * B, B)], sem),
        pltpu.async_copy(mx_hbm, pvec.at[pl.ds(3 * B, B)], sem),
    ]

    zero = jnp.zeros((L,), jnp.float32)
    for j in range(PAD // L):
        h0[pl.ds(j * L, L)] = zero
        h1[pl.ds(j * L, L)] = zero

    for cp in cps:
        cp.wait()
    modv[pl.ds(N, L)] = zero
    modv[pl.ds(N + L, L)] = zero

    bvec = jnp.full((L,), b, jnp.int32)
    fb = plsc.load_gather(pvec, [bvec])
    wd = plsc.load_gather(pvec, [bvec + B])
    dp = plsc.load_gather(pvec, [bvec + 2 * B])
    mx = plsc.load_gather(pvec, [bvec + 3 * B])
    cw = jnp.float32(MAX_LFO) * wd
    omx = jnp.float32(1.0) - mx

    lanes = lax.iota(jnp.int32, L)
    fD = jnp.float32(D)
    fMIN = jnp.float32(MIN_D)

    def idxmath(off, wq0):
        tp = (off + PAD) + lanes
        wqv = wq0 + lanes
        wqv = jnp.where(wqv >= D, wqv - D, wqv)
        wf = wqv.astype(jnp.float32)
        mv = modv[pl.ds(pl.multiple_of(off, L), L)]
        delay = cw * mv + fMIN
        a = wf - delay + fD
        rid = jnp.where(a >= fD, a - fD, a)
        piv = rid.astype(jnp.int32)
        fr = rid - piv.astype(jnp.float32)
        dist = wqv - piv
        dist = jnp.where(dist <= 0, dist + D, dist)
        wq1 = wq0 + L
        wq1 = jnp.where(wq1 >= D, wq1 - D, wq1)
        return tp - dist, fr, wq1

    def gather4(gi):
        gn = gi + 1
        return (plsc.load_gather(h0, [gi]), plsc.load_gather(h0, [gn]),
                plsc.load_gather(h1, [gi]), plsc.load_gather(h1, [gn]))

    g0, f0, wq1 = idxmath(jnp.int32(0), jnp.int32(0))
    g1, f1, wq2 = idxmath(jnp.int32(L), wq1)
    p0, n0, p1, n1 = gather4(g0)

    def step(j, carry):
        frc, pv0, nv0, pv1, nv1, gnx, fnx, wqn = carry
        off = j * L
        q0, m0, q1, m1 = gather4(gnx)
        om = jnp.float32(1.0) - frc
        for (xr, hr, orr, pv, nv) in ((x0, h0, o0, pv0, nv0),
                                      (x1, h1, o1, pv1, nv1)):
            iv = frc * nv + om * pv
            xv = xr[pl.ds(pl.multiple_of(off, L), L)]
            hr[pl.ds(pl.multiple_of(off + PAD, L), L)] = xv + fb * iv
            ov = xv + dp * iv
            orr[pl.ds(pl.multiple_of(off, L), L)] = omx * xv + mx * ov
        g2, f2, wq3 = idxmath(off + 2 * L, wqn)
        return (fnx, q0, m0, q1, m1, g2, f2, wq3)

    lax.fori_loop(0, NG, step, (f0, p0, n0, p1, n1, g1, f1, wq2))

    pltpu.sync_copy(o0, out_hbm.at[b, 0])
    pltpu.sync_copy(o1, out_hbm.at[b, 1])


@jax.jit
def kernel(x, mod_sig, feedback, width, depth, mix):
    mesh = plsc.VectorSubcoreMesh(core_axis_name="c", subcore_axis_name="s")
    f = functools.partial(
        pl.kernel,
        mesh=mesh,
        compiler_params=pltpu.CompilerParams(needs_layout_passes=False),
        out_type=jax.ShapeDtypeStruct((B, C, N), jnp.float32),
        scratch_types=[
            pltpu.VMEM((N + 2 * L,), jnp.float32),
            pltpu.VMEM((N,), jnp.float32),
            pltpu.VMEM((N,), jnp.float32),
            pltpu.VMEM((PAD + N,), jnp.float32),
            pltpu.VMEM((PAD + N,), jnp.float32),
            pltpu.VMEM((N,), jnp.float32),
            pltpu.VMEM((N,), jnp.float32),
            pltpu.VMEM((4 * B,), jnp.float32),
            pltpu.SemaphoreType.DMA,
        ],
    )(_flanger_body)
    return f(x, mod_sig, feedback, width, depth, mix)


# speedup vs baseline: 1574.3232x; 1.0219x over previous
import functools

import jax
import jax.numpy as jnp
from jax import lax
from jax.experimental import pallas as pl
from jax.experimental.pallas import tpu as pltpu
from jax.experimental.pallas import tpu_sc as plsc

B = 32
C = 2
N = 8192
D = 485
MIN_D = 44
MAX_LFO = 441
PAD = 496
L = 16
NG = N // L


def _flanger_body(x_hbm, mod_hbm, fb_hbm, w_hbm, dp_hbm, mx_hbm, out_hbm,
                  modv, x0, x1, h0, h1, o0, o1, pvec, sem):
    b = lax.axis_index("s") * 2 + lax.axis_index("c")

    cps = [
        pltpu.async_copy(mod_hbm.at[b], modv.at[pl.ds(0, N)], sem),
        pltpu.async_copy(x_hbm.at[b, 0], x0, sem),
        pltpu.async_copy(x_hbm.at[b, 1], x1, sem),
        pltpu.async_copy(fb_hbm, pvec.at[pl.ds(0, B)], sem),
        pltpu.async_copy(w_hbm, pvec.at[pl.ds(B, B)], sem),
        pltpu.async_copy(dp_hbm, pvec.at[pl.ds(2 * B, B)], sem),
        pltpu.async_copy(mx_hbm, pvec.at[pl.ds(3 * B, B)], sem),
    ]

    zero = jnp.zeros((L,), jnp.float32)
    for j in range(PAD // L):
        h0[pl.ds(j * L, L)] = zero
        h1[pl.ds(j * L, L)] = zero

    for cp in cps:
        cp.wait()
    modv[pl.ds(N, L)] = zero
    modv[pl.ds(N + L, L)] = zero

    bvec = jnp.full((L,), b, jnp.int32)
    fb = plsc.load_gather(pvec, [bvec])
    wd = plsc.load_gather(pvec, [bvec + B])
    dp = plsc.load_gather(pvec, [bvec + 2 * B])
    mx = plsc.load_gather(pvec, [bvec + 3 * B])
    cw = jnp.float32(MAX_LFO) * wd
    mdp = mx * dp

    lanes = lax.iota(jnp.int32, L)
    fD = jnp.float32(D)
    fMIN = jnp.float32(MIN_D)

    def idxmath(off, wq0):
        tp = (off + PAD - D) + lanes
        wqv = wq0 + lanes
        wqv = jnp.where(wqv >= D, wqv - D, wqv)
        wf = wqv.astype(jnp.float32)
        mv = modv[pl.ds(pl.multiple_of(off, L), L)]
        delay = cw * mv + fMIN
        a = wf - delay + fD
        pa = a.astype(jnp.int32)
        fr = a - pa.astype(jnp.float32)
        wq1 = wq0 + L
        wq1 = jnp.where(wq1 >= D, wq1 - D, wq1)
        return tp - wqv + pa, fr, wq1

    def gather4(gi):
        gn = gi + 1
        return (plsc.load_gather(h0, [gi]), plsc.load_gather(h0, [gn]),
                plsc.load_gather(h1, [gi]), plsc.load_gather(h1, [gn]))

    def emit_group(off, frc, taps):
        pv0, nv0, pv1, nv1 = taps
        om = jnp.float32(1.0) - frc
        for (xr, hr, orr, pv, nv) in ((x0, h0, o0, pv0, nv0),
                                      (x1, h1, o1, pv1, nv1)):
            iv = frc * nv + om * pv
            xv = xr[pl.ds(pl.multiple_of(off, L), L)]
            hr[pl.ds(pl.multiple_of(off + PAD, L), L)] = xv + fb * iv
            orr[pl.ds(pl.multiple_of(off, L), L)] = xv + mdp * iv

    g0, f0, wq1 = idxmath(jnp.int32(0), jnp.int32(0))
    g1, f1, wq2 = idxmath(jnp.int32(L), wq1)
    taps0 = gather4(g0)

    def step(jj, carry):
        fra, ta0, ta1, ta2, ta3, gb, frb, wqc = carry
        off_a = jj * (2 * L)
        taps_b = gather4(gb)
        emit_group(off_a, fra, (ta0, ta1, ta2, ta3))
        gc, frc, wqd = idxmath(off_a + 2 * L, wqc)
        taps_c = gather4(gc)
        emit_group(off_a + L, frb, taps_b)
        gd, frd, wqe = idxmath(off_a + 3 * L, wqd)
        return (frc, *taps_c, gd, frd, wqe)

    lax.fori_loop(0, NG // 2, step, (f0, *taps0, g1, f1, wq2))

    pltpu.sync_copy(o0, out_hbm.at[b, 0])
    pltpu.sync_copy(o1, out_hbm.at[b, 1])


@jax.jit
def kernel(x, mod_sig, feedback, width, depth, mix):
    mesh = plsc.VectorSubcoreMesh(core_axis_name="c", subcore_axis_name="s")
    f = functools.partial(
        pl.kernel,
        mesh=mesh,
        compiler_params=pltpu.CompilerParams(needs_layout_passes=False),
        out_type=jax.ShapeDtypeStruct((B, C, N), jnp.float32),
        scratch_types=[
            pltpu.VMEM((N + 2 * L,), jnp.float32),
            pltpu.VMEM((N,), jnp.float32),
            pltpu.VMEM((N,), jnp.float32),
            pltpu.VMEM((PAD + N,), jnp.float32),
            pltpu.VMEM((PAD + N,), jnp.float32),
            pltpu.VMEM((N,), jnp.float32),
            pltpu.VMEM((N,), jnp.float32),
            pltpu.VMEM((4 * B,), jnp.float32),
            pltpu.SemaphoreType.DMA,
        ],
    )(_flanger_body)
    return f(x, mod_sig, feedback, width, depth, mix)
